# Initial kernel scaffold; baseline (speedup 1.0000x reference)
import jax
import jax.numpy as jnp
from jax.experimental import pallas as pl


def kernel(x, edge_index, edge_weight, W_conv, b_conv, prelu_a, W_cat, b_cat):
    raise NotImplementedError("write your pallas kernel here")


# speedup vs baseline: 4.5851x; 4.5851x over previous
import functools

import jax
import jax.numpy as jnp
from jax import lax
from jax.experimental import pallas as pl
from jax.experimental.pallas import tpu as pltpu
from jax.experimental.pallas import tpu_sc as plsc

N_NODES = 10000
N_EDGES = 320000
D = 128
NC = 2
NS = 16
NW = NC * NS
E_PER_W = N_EDGES // NW
CHUNK = 80
NCHUNK = E_PER_W // CHUNK
ROWS_PER_TILE = 624
ROWS_TAIL = N_NODES - NS * ROWS_PER_TILE
ZROWS = 104


def _sc_segment_sum(x, src, dst, ew):
    mesh = plsc.VectorSubcoreMesh(core_axis_name="c", subcore_axis_name="s",
                                  num_cores=NC, num_subcores=NS)

    @functools.partial(
        pl.kernel,
        out_type=jax.ShapeDtypeStruct((NC, N_NODES, D), jnp.float32),
        mesh=mesh,
        scratch_types=[
            pltpu.VMEM((CHUNK,), jnp.int32),
            pltpu.VMEM((CHUNK,), jnp.int32),
            pltpu.VMEM((CHUNK,), jnp.float32),
            pltpu.VMEM((CHUNK, D), jnp.float32),
            pltpu.VMEM((ZROWS, D), jnp.float32),
            pltpu.VMEM_SHARED((N_NODES, D), jnp.float32),
            pltpu.SemaphoreType.DMA,
        ],
    )
    def k(x_hbm, src_hbm, dst_hbm, ew_hbm, out_hbm,
          src_v, dst_v, ew_v, rows_v, zero_v, acc_sh, sem):
        cid = lax.axis_index("c")
        sid = lax.axis_index("s")
        wid = sid * NC + cid

        zvec = jnp.zeros((16,), jnp.float32)

        def zrow(i, carry):
            for j in range(D // 16):
                zero_v[i, pl.ds(j * 16, 16)] = zvec
            return carry
        lax.fori_loop(0, ZROWS, zrow, 0)
        for b in range(ROWS_PER_TILE // ZROWS):
            pltpu.sync_copy(zero_v,
                            acc_sh.at[pl.ds(sid * ROWS_PER_TILE + b * ZROWS, ZROWS)])

        @pl.when(sid == NS - 1)
        def _zero_tail():
            pltpu.sync_copy(zero_v.at[pl.ds(0, ROWS_TAIL)],
                            acc_sh.at[pl.ds(NS * ROWS_PER_TILE, ROWS_TAIL)])
        plsc.subcore_barrier()

        base = pl.multiple_of(wid * E_PER_W, 8)

        def chunk(i, carry):
            off = pl.multiple_of(base + i * CHUNK, 8)
            pltpu.sync_copy(src_hbm.at[pl.ds(off, CHUNK)], src_v)
            pltpu.sync_copy(dst_hbm.at[pl.ds(off, CHUNK)], dst_v)
            pltpu.sync_copy(ew_hbm.at[pl.ds(off, CHUNK)], ew_v)
            pltpu.async_copy(x_hbm.at[src_v], rows_v, sem).wait()

            def scale(g, c2):
                wv = ew_v[pl.ds(g * 16, 16)]
                dnums = lax.GatherDimensionNumbers(
                    offset_dims=(), collapsed_slice_dims=(0,),
                    start_index_map=(0,))
                for l in range(16):
                    wl = lax.gather(
                        wv, jnp.full((16, 1), l, jnp.int32), dnums,
                        slice_sizes=(1,),
                        mode=lax.GatherScatterMode.PROMISE_IN_BOUNDS)
                    e = g * 16 + l
                    for j in range(D // 16):
                        rows_v[e, pl.ds(j * 16, 16)] = (
                            rows_v[e, pl.ds(j * 16, 16)] * wl)
                return c2
            lax.fori_loop(0, CHUNK // 16, scale, 0)
            pltpu.sync_copy(rows_v, acc_sh.at[dst_v], add=True)
            return carry
        lax.fori_loop(0, NCHUNK, chunk, 0)
        plsc.subcore_barrier()

        r0 = pl.multiple_of(sid * ROWS_PER_TILE, 8)
        pltpu.sync_copy(acc_sh.at[pl.ds(r0, ROWS_PER_TILE)],
                        out_hbm.at[cid, pl.ds(r0, ROWS_PER_TILE)])

        @pl.when(sid == NS - 1)
        def _copy_tail():
            pltpu.sync_copy(acc_sh.at[pl.ds(NS * ROWS_PER_TILE, ROWS_TAIL)],
                            out_hbm.at[cid, pl.ds(NS * ROWS_PER_TILE, ROWS_TAIL)])

    return k(x, src, dst, ew)


def _tc_body(pr_ref, wct_ref, bc_ref, pa_ref, w1_ref, w2_ref, bk_ref, y_ref):
    s0 = pr_ref[0, 0] + pr_ref[1, 0]
    s1 = pr_ref[0, 1] + pr_ref[1, 1]
    a = jnp.dot(s0, wct_ref[...], preferred_element_type=jnp.float32) + bc_ref[...]
    b = jnp.dot(s1, wct_ref[...], preferred_element_type=jnp.float32) + bc_ref[...]
    pa = pa_ref[...]
    a = jnp.where(a >= 0, a, a * pa)
    b = jnp.where(b >= 0, b, b * pa)
    y = (jnp.dot(a, w1_ref[...], preferred_element_type=jnp.float32)
         + jnp.dot(b, w2_ref[...], preferred_element_type=jnp.float32)
         + bk_ref[...])
    y_ref[...] = jnp.maximum(y, 0.0)


def kernel(x, edge_index, edge_weight, W_conv, b_conv, prelu_a, W_cat, b_cat):
    src = edge_index[0].astype(jnp.int32)
    dst = edge_index[1].astype(jnp.int32)
    ew = edge_weight.astype(jnp.float32)

    partials = _sc_segment_sum(x, src, dst, ew)
    n = N_NODES // 2
    pr = partials.reshape(NC, 2, n, D)

    wct = W_conv.T
    w1 = W_cat[:, :D].T
    w2 = W_cat[:, D:].T
    bc = b_conv.reshape(1, D)
    pa = prelu_a.reshape(1, D)
    bk = b_cat.reshape(1, D)

    BS = 1000
    grid = (n // BS,)
    y = pl.pallas_call(
        _tc_body,
        grid=grid,
        in_specs=[
            pl.BlockSpec((NC, 2, BS, D), lambda i: (0, 0, i, 0)),
            pl.BlockSpec((D, D), lambda i: (0, 0)),
            pl.BlockSpec((1, D), lambda i: (0, 0)),
            pl.BlockSpec((1, D), lambda i: (0, 0)),
            pl.BlockSpec((D, D), lambda i: (0, 0)),
            pl.BlockSpec((D, D), lambda i: (0, 0)),
            pl.BlockSpec((1, D), lambda i: (0, 0)),
        ],
        out_specs=pl.BlockSpec((BS, D), lambda i: (i, 0)),
        out_shape=jax.ShapeDtypeStruct((n, D), jnp.float32),
    )(pr, wct, bc, pa, w1, w2, bk)
    return y
